# baseline (speedup 1.0000x reference)
import jax
import jax.numpy as jnp
from jax import lax
from jax.experimental import pallas as pl
from jax.experimental.pallas import tpu as pltpu
from jax.experimental.pallas import tpu_sc as plsc

NC = 2
NS = 16
NW = NC * NS
D = 128
BN = 1000


def _edge_chunk(per_w, maxk=128):
    for k in range(maxk, 0, -8):
        if per_w % k == 0:
            return k
    raise ValueError(per_w)


def _round_up(v, m):
    return (v + m - 1) // m * m


def _sc_degree(dstp, n):
    nck = dstp.shape[1]
    ck = dstp.shape[2]
    rpt = _round_up(-(-n // NS), 8)
    np_ = rpt * NS
    win = 4
    mesh = plsc.VectorSubcoreMesh(core_axis_name="c", subcore_axis_name="s")

    def body(dstp_h, zeros_h, ones_h, out_h, idx_v, ones_v, acc_sh, sem):
        c = lax.axis_index("c")
        s = lax.axis_index("s")
        wid = s * NC + c

        base_r = s * rpt
        pltpu.sync_copy(zeros_h, acc_sh.at[pl.ds(base_r, rpt)])
        pltpu.sync_copy(ones_h, ones_v)
        pltpu.sync_copy(dstp_h.at[wid], idx_v)
        plsc.subcore_barrier()

        def swait():
            pltpu.make_async_copy(ones_v, acc_sh.at[pl.ds(0, ck)], sem).wait()

        def step(i, _):
            pltpu.async_copy(ones_v, acc_sh.at[idx_v.at[i]], sem, add=True)

            @pl.when(i >= win)
            def _():
                swait()
            return 0
        lax.fori_loop(0, nck, step, 0)
        for _ in range(min(win, nck)):
            swait()
        plsc.subcore_barrier()
        pltpu.sync_copy(acc_sh.at[pl.ds(base_r, rpt)],
                        out_h.at[pl.ds(c * np_ + base_r, rpt)])

    return pl.kernel(
        body,
        out_type=jax.ShapeDtypeStruct((NC * np_, D), jnp.float32),
        mesh=mesh,
        scratch_types=[
            pltpu.VMEM((nck, ck), jnp.int32),
            pltpu.VMEM((ck, D), jnp.float32),
            pltpu.VMEM_SHARED((np_, D), jnp.float32),
            pltpu.SemaphoreType.DMA,
        ],
    )(dstp, jnp.zeros((rpt, D), jnp.float32),
      jnp.ones((ck, D), jnp.float32)), np_


def _sc_scatter(table, srcp, dstp):
    n = table.shape[0]
    nck = srcp.shape[1]
    ck = srcp.shape[2]
    rpt = _round_up(-(-n // NS), 8)
    np_ = rpt * NS
    assert n < np_
    PH = 2
    L = -(-nck // PH)
    mesh = plsc.VectorSubcoreMesh(core_axis_name="c", subcore_axis_name="s")

    def body(table_h, src_h, dst_h, zeros_h, out_h, idx_s, idx_d,
             rows_a, rows_b, acc_sh, gsem_a, gsem_b, ssem_a, ssem_b):
        c = lax.axis_index("c")
        s = lax.axis_index("s")
        wid = s * NC + c

        base_r = s * rpt
        pltpu.sync_copy(zeros_h, acc_sh.at[pl.ds(base_r, rpt)])
        plsc.subcore_barrier()

        def gather(i, buf, sem):
            pltpu.async_copy(table_h.at[idx_s.at[i]], buf, sem)

        def gwait(buf, sem):
            pltpu.make_async_copy(table_h.at[pl.ds(0, ck)], buf, sem).wait()

        def sstart(i, buf, sem):
            pltpu.async_copy(buf, acc_sh.at[idx_d.at[i]], sem, add=True)

        def swait(buf, sem):
            pltpu.make_async_copy(buf, acc_sh.at[pl.ds(0, ck)], sem).wait()

        for p in range(PH):
            cbase = p * L
            lp = min(L, nck - cbase)
            if lp <= 0:
                continue
            pltpu.sync_copy(src_h.at[wid, pl.ds(cbase, lp)],
                            idx_s.at[pl.ds(0, lp)])
            pltpu.sync_copy(dst_h.at[wid, pl.ds(cbase, lp)],
                            idx_d.at[pl.ds(0, lp)])

            gather(0, rows_a, gsem_a)
            if lp > 1:
                gather(1, rows_b, gsem_b)

            def pair(k, _):
                i = 2 * k
                gwait(rows_a, gsem_a)
                sstart(i, rows_a, ssem_a)
                gwait(rows_b, gsem_b)
                sstart(i + 1, rows_b, ssem_b)
                swait(rows_a, ssem_a)

                @pl.when(i + 2 < lp)
                def _():
                    gather(i + 2, rows_a, gsem_a)
                swait(rows_b, ssem_b)

                @pl.when(i + 3 < lp)
                def _():
                    gather(i + 3, rows_b, gsem_b)
                return 0
            lax.fori_loop(0, lp // 2, pair, 0)
            if lp % 2:
                gwait(rows_a, gsem_a)
                pltpu.sync_copy(rows_a, acc_sh.at[idx_d.at[lp - 1]], add=True)

        plsc.subcore_barrier()
        pltpu.sync_copy(acc_sh.at[pl.ds(base_r, rpt)],
                        out_h.at[pl.ds(c * np_ + base_r, rpt)])

    return pl.kernel(
        body,
        out_type=jax.ShapeDtypeStruct((NC * np_, D), jnp.float32),
        mesh=mesh,
        scratch_types=[
            pltpu.VMEM((L, ck), jnp.int32),
            pltpu.VMEM((L, ck), jnp.int32),
            pltpu.VMEM((ck, D), jnp.float32),
            pltpu.VMEM((ck, D), jnp.float32),
            pltpu.VMEM_SHARED((np_, D), jnp.float32),
            pltpu.SemaphoreType.DMA,
            pltpu.SemaphoreType.DMA,
            pltpu.SemaphoreType.DMA,
            pltpu.SemaphoreType.DMA,
        ],
    )(table, srcp, dstp, jnp.zeros((rpt, D), jnp.float32)), np_


def _tc_stage1(x, w1, d0, d1):
    n = x.shape[0]

    def body(x_r, w_r, d0_r, d1_r, h_r, hh_r, dinv_r):
        deg = d0_r[...] + d1_r[...] + 1.0
        dinv = lax.rsqrt(deg)
        h = jnp.dot(x_r[...], w_r[...], preferred_element_type=jnp.float32)
        h_r[...] = h
        hh_r[...] = h * dinv
        dinv_r[...] = dinv

    return pl.pallas_call(
        body,
        grid=(n // BN,),
        in_specs=[
            pl.BlockSpec((BN, D), lambda i: (i, 0)),
            pl.BlockSpec((D, D), lambda i: (0, 0)),
            pl.BlockSpec((BN, 1), lambda i: (i, 0)),
            pl.BlockSpec((BN, 1), lambda i: (i, 0)),
        ],
        out_specs=[
            pl.BlockSpec((BN, D), lambda i: (i, 0)),
            pl.BlockSpec((BN, D), lambda i: (i, 0)),
            pl.BlockSpec((BN, 1), lambda i: (i, 0)),
        ],
        out_shape=[
            jax.ShapeDtypeStruct((n, D), jnp.float32),
            jax.ShapeDtypeStruct((n, D), jnp.float32),
            jax.ShapeDtypeStruct((n, 1), jnp.float32),
        ],
    )(x, w1, d0, d1)


def _tc_stage2(a0, a1, h1, dinv, w2, b1):
    n = h1.shape[0]

    def body(a0_r, a1_r, h1_r, dinv_r, w_r, b_r, h2_r, hh2_r):
        dinv = dinv_r[...]
        t = (a0_r[...] + a1_r[...]) * dinv + h1_r[...] * (dinv * dinv) + b_r[...]
        t = jnp.maximum(t, 0.0)
        h2 = jnp.dot(t, w_r[...], preferred_element_type=jnp.float32)
        h2_r[...] = h2
        hh2_r[...] = h2 * dinv

    return pl.pallas_call(
        body,
        grid=(n // BN,),
        in_specs=[
            pl.BlockSpec((BN, D), lambda i: (i, 0)),
            pl.BlockSpec((BN, D), lambda i: (i, 0)),
            pl.BlockSpec((BN, D), lambda i: (i, 0)),
            pl.BlockSpec((BN, 1), lambda i: (i, 0)),
            pl.BlockSpec((D, D), lambda i: (0, 0)),
            pl.BlockSpec((1, D), lambda i: (0, 0)),
        ],
        out_specs=[
            pl.BlockSpec((BN, D), lambda i: (i, 0)),
            pl.BlockSpec((BN, D), lambda i: (i, 0)),
        ],
        out_shape=[
            jax.ShapeDtypeStruct((n, D), jnp.float32),
            jax.ShapeDtypeStruct((n, D), jnp.float32),
        ],
    )(a0, a1, h1, dinv, w2, b1.reshape(1, D))


def _tc_stage3(a0, a1, h2, dinv, b2):
    n = h2.shape[0]

    def body(a0_r, a1_r, h2_r, dinv_r, b_r, o_r):
        dinv = dinv_r[...]
        o_r[...] = ((a0_r[...] + a1_r[...]) * dinv
                    + h2_r[...] * (dinv * dinv) + b_r[...])

    return pl.pallas_call(
        body,
        grid=(n // BN,),
        in_specs=[
            pl.BlockSpec((BN, D), lambda i: (i, 0)),
            pl.BlockSpec((BN, D), lambda i: (i, 0)),
            pl.BlockSpec((BN, D), lambda i: (i, 0)),
            pl.BlockSpec((BN, 1), lambda i: (i, 0)),
            pl.BlockSpec((1, D), lambda i: (0, 0)),
        ],
        out_specs=pl.BlockSpec((BN, D), lambda i: (i, 0)),
        out_shape=jax.ShapeDtypeStruct((n, D), jnp.float32),
    )(a0, a1, h2, dinv, b2.reshape(1, D))


def kernel(x, edge_index, W1, b1, W2, b2):
    n = x.shape[0]
    e = edge_index.shape[1]
    per_w = e // NW
    ck = 128
    nck = -(-per_w // ck)
    pad = nck * ck - per_w
    srcp = jnp.pad(edge_index[0].reshape(NW, per_w),
                   ((0, 0), (0, pad))).reshape(NW, nck, ck)
    dstp = jnp.pad(edge_index[1].reshape(NW, per_w), ((0, 0), (0, pad)),
                   constant_values=n).reshape(NW, nck, ck)

    degp, np_ = _sc_degree(dstp, n)
    d0 = degp[:n, 0:1]
    d1 = degp[np_:np_ + n, 0:1]
    h1, hh1, dinv = _tc_stage1(x, W1, d0, d1)
    agg1, _ = _sc_scatter(hh1, srcp, dstp)
    h2, hh2 = _tc_stage2(agg1[:n], agg1[np_:np_ + n], h1, dinv, W2, b1)
    agg2, _ = _sc_scatter(hh2, srcp, dstp)
    return _tc_stage3(agg2[:n], agg2[np_:np_ + n], h2, dinv, b2)


# speedup vs baseline: 1.1077x; 1.1077x over previous
import jax
import jax.numpy as jnp
from jax import lax
from jax.experimental import pallas as pl
from jax.experimental.pallas import tpu as pltpu
from jax.experimental.pallas import tpu_sc as plsc

NC = 2
NS = 16
NW = NC * NS
D = 128
BN = 1000


def _edge_chunk(per_w, maxk=128):
    for k in range(maxk, 0, -8):
        if per_w % k == 0:
            return k
    raise ValueError(per_w)


def _round_up(v, m):
    return (v + m - 1) // m * m


def _sc_degree(dstp, n):
    nck = dstp.shape[1]
    ck = dstp.shape[2]
    rpt = _round_up(-(-n // NS), 8)
    np_ = rpt * NS
    win = 4
    mesh = plsc.VectorSubcoreMesh(core_axis_name="c", subcore_axis_name="s")

    def body(dstp_h, zeros_h, ones_h, out_h, idx_v, ones_v, acc_sh, sem):
        c = lax.axis_index("c")
        s = lax.axis_index("s")
        wid = s * NC + c

        base_r = s * rpt
        pltpu.sync_copy(zeros_h, acc_sh.at[pl.ds(base_r, rpt)])
        pltpu.sync_copy(ones_h, ones_v)
        pltpu.sync_copy(dstp_h.at[wid], idx_v)
        plsc.subcore_barrier()

        def swait():
            pltpu.make_async_copy(ones_v, acc_sh.at[pl.ds(0, ck)], sem).wait()

        def step(i, _):
            pltpu.async_copy(ones_v, acc_sh.at[idx_v.at[i]], sem, add=True)

            @pl.when(i >= win)
            def _():
                swait()
            return 0
        lax.fori_loop(0, nck, step, 0)
        for _ in range(min(win, nck)):
            swait()
        plsc.subcore_barrier()
        pltpu.sync_copy(acc_sh.at[pl.ds(base_r, rpt)],
                        out_h.at[pl.ds(c * np_ + base_r, rpt)])

    return pl.kernel(
        body,
        out_type=jax.ShapeDtypeStruct((NC * np_, D), jnp.float32),
        mesh=mesh,
        scratch_types=[
            pltpu.VMEM((nck, ck), jnp.int32),
            pltpu.VMEM((ck, D), jnp.float32),
            pltpu.VMEM_SHARED((np_, D), jnp.float32),
            pltpu.SemaphoreType.DMA,
        ],
    )(dstp, jnp.zeros((rpt, D), jnp.float32),
      jnp.ones((ck, D), jnp.float32)), np_


def _sc_scatter(table, srcp, dstp):
    n = table.shape[0]
    nck = srcp.shape[1]
    ck = srcp.shape[2]
    rpt = _round_up(-(-n // NS), 8)
    np_ = rpt * NS
    assert n < np_
    PH = 2
    L = -(-nck // PH)
    mesh = plsc.VectorSubcoreMesh(core_axis_name="c", subcore_axis_name="s")

    def body(table_h, src_h, dst_h, zeros_h, out_h, idx_s, idx_d,
             rows_a, rows_b, acc_sh, gsem_a, gsem_b):
        c = lax.axis_index("c")
        s = lax.axis_index("s")
        wid = s * NC + c

        base_r = s * rpt
        pltpu.sync_copy(zeros_h, acc_sh.at[pl.ds(base_r, rpt)])

        def gather(i, buf, sem):
            pltpu.async_copy(table_h.at[idx_s.at[i]], buf, sem)

        def gwait(buf, sem):
            pltpu.make_async_copy(table_h.at[pl.ds(0, ck)], buf, sem).wait()

        for p in range(PH):
            cbase = p * L
            lp = min(L, nck - cbase)
            if lp <= 0:
                continue
            pltpu.sync_copy(src_h.at[wid, pl.ds(cbase, lp)],
                            idx_s.at[pl.ds(0, lp)])
            pltpu.sync_copy(dst_h.at[wid, pl.ds(cbase, lp)],
                            idx_d.at[pl.ds(0, lp)])

            gather(0, rows_a, gsem_a)
            if lp > 1:
                gather(1, rows_b, gsem_b)
            if p == 0:
                plsc.subcore_barrier()

            def pair(k, _):
                i = 2 * k
                gwait(rows_a, gsem_a)
                pltpu.sync_copy(rows_a, acc_sh.at[idx_d.at[i]], add=True)

                @pl.when(i + 2 < lp)
                def _():
                    gather(i + 2, rows_a, gsem_a)
                gwait(rows_b, gsem_b)
                pltpu.sync_copy(rows_b, acc_sh.at[idx_d.at[i + 1]], add=True)

                @pl.when(i + 3 < lp)
                def _():
                    gather(i + 3, rows_b, gsem_b)
                return 0
            lax.fori_loop(0, lp // 2, pair, 0)
            if lp % 2:
                gwait(rows_a, gsem_a)
                pltpu.sync_copy(rows_a, acc_sh.at[idx_d.at[lp - 1]], add=True)

        plsc.subcore_barrier()
        pltpu.sync_copy(acc_sh.at[pl.ds(base_r, rpt)],
                        out_h.at[pl.ds(c * np_ + base_r, rpt)])

    return pl.kernel(
        body,
        out_type=jax.ShapeDtypeStruct((NC * np_, D), jnp.float32),
        mesh=mesh,
        scratch_types=[
            pltpu.VMEM((L, ck), jnp.int32),
            pltpu.VMEM((L, ck), jnp.int32),
            pltpu.VMEM((ck, D), jnp.float32),
            pltpu.VMEM((ck, D), jnp.float32),
            pltpu.VMEM_SHARED((np_, D), jnp.float32),
            pltpu.SemaphoreType.DMA,
            pltpu.SemaphoreType.DMA,
        ],
    )(table, srcp, dstp, jnp.zeros((rpt, D), jnp.float32)), np_


def _tc_stage1(x, w1, d0, d1):
    n = x.shape[0]

    def body(x_r, w_r, d0_r, d1_r, h_r, hh_r, dinv_r):
        deg = d0_r[...] + d1_r[...] + 1.0
        dinv = lax.rsqrt(deg)
        h = jnp.dot(x_r[...], w_r[...], preferred_element_type=jnp.float32)
        h_r[...] = h
        hh_r[...] = h * dinv
        dinv_r[...] = dinv

    return pl.pallas_call(
        body,
        grid=(n // BN,),
        in_specs=[
            pl.BlockSpec((BN, D), lambda i: (i, 0)),
            pl.BlockSpec((D, D), lambda i: (0, 0)),
            pl.BlockSpec((BN, 1), lambda i: (i, 0)),
            pl.BlockSpec((BN, 1), lambda i: (i, 0)),
        ],
        out_specs=[
            pl.BlockSpec((BN, D), lambda i: (i, 0)),
            pl.BlockSpec((BN, D), lambda i: (i, 0)),
            pl.BlockSpec((BN, 1), lambda i: (i, 0)),
        ],
        out_shape=[
            jax.ShapeDtypeStruct((n, D), jnp.float32),
            jax.ShapeDtypeStruct((n, D), jnp.float32),
            jax.ShapeDtypeStruct((n, 1), jnp.float32),
        ],
    )(x, w1, d0, d1)


def _tc_stage2(a0, a1, h1, dinv, w2, b1):
    n = h1.shape[0]

    def body(a0_r, a1_r, h1_r, dinv_r, w_r, b_r, h2_r, hh2_r):
        dinv = dinv_r[...]
        t = (a0_r[...] + a1_r[...]) * dinv + h1_r[...] * (dinv * dinv) + b_r[...]
        t = jnp.maximum(t, 0.0)
        h2 = jnp.dot(t, w_r[...], preferred_element_type=jnp.float32)
        h2_r[...] = h2
        hh2_r[...] = h2 * dinv

    return pl.pallas_call(
        body,
        grid=(n // BN,),
        in_specs=[
            pl.BlockSpec((BN, D), lambda i: (i, 0)),
            pl.BlockSpec((BN, D), lambda i: (i, 0)),
            pl.BlockSpec((BN, D), lambda i: (i, 0)),
            pl.BlockSpec((BN, 1), lambda i: (i, 0)),
            pl.BlockSpec((D, D), lambda i: (0, 0)),
            pl.BlockSpec((1, D), lambda i: (0, 0)),
        ],
        out_specs=[
            pl.BlockSpec((BN, D), lambda i: (i, 0)),
            pl.BlockSpec((BN, D), lambda i: (i, 0)),
        ],
        out_shape=[
            jax.ShapeDtypeStruct((n, D), jnp.float32),
            jax.ShapeDtypeStruct((n, D), jnp.float32),
        ],
    )(a0, a1, h1, dinv, w2, b1.reshape(1, D))


def _tc_stage3(a0, a1, h2, dinv, b2):
    n = h2.shape[0]

    def body(a0_r, a1_r, h2_r, dinv_r, b_r, o_r):
        dinv = dinv_r[...]
        o_r[...] = ((a0_r[...] + a1_r[...]) * dinv
                    + h2_r[...] * (dinv * dinv) + b_r[...])

    return pl.pallas_call(
        body,
        grid=(n // BN,),
        in_specs=[
            pl.BlockSpec((BN, D), lambda i: (i, 0)),
            pl.BlockSpec((BN, D), lambda i: (i, 0)),
            pl.BlockSpec((BN, D), lambda i: (i, 0)),
            pl.BlockSpec((BN, 1), lambda i: (i, 0)),
            pl.BlockSpec((1, D), lambda i: (0, 0)),
        ],
        out_specs=pl.BlockSpec((BN, D), lambda i: (i, 0)),
        out_shape=jax.ShapeDtypeStruct((n, D), jnp.float32),
    )(a0, a1, h2, dinv, b2.reshape(1, D))


def kernel(x, edge_index, W1, b1, W2, b2):
    n = x.shape[0]
    e = edge_index.shape[1]
    per_w = e // NW
    ck = 128
    nck = -(-per_w // ck)
    pad = nck * ck - per_w
    srcp = jnp.pad(edge_index[0].reshape(NW, per_w),
                   ((0, 0), (0, pad))).reshape(NW, nck, ck)
    dstp = jnp.pad(edge_index[1].reshape(NW, per_w), ((0, 0), (0, pad)),
                   constant_values=n).reshape(NW, nck, ck)

    degp, np_ = _sc_degree(dstp, n)
    d0 = degp[:n, 0:1]
    d1 = degp[np_:np_ + n, 0:1]
    h1, hh1, dinv = _tc_stage1(x, W1, d0, d1)
    agg1, _ = _sc_scatter(hh1, srcp, dstp)
    h2, hh2 = _tc_stage2(agg1[:n], agg1[np_:np_ + n], h1, dinv, W2, b1)
    agg2, _ = _sc_scatter(hh2, srcp, dstp)
    return _tc_stage3(agg2[:n], agg2[np_:np_ + n], h2, dinv, b2)


# speedup vs baseline: 1.1185x; 1.0097x over previous
import jax
import jax.numpy as jnp
from jax import lax
from jax.experimental import pallas as pl
from jax.experimental.pallas import tpu as pltpu
from jax.experimental.pallas import tpu_sc as plsc

NC = 2
NS = 16
NW = NC * NS
D = 128
BN = 1000


def _edge_chunk(per_w, maxk=128):
    for k in range(maxk, 0, -8):
        if per_w % k == 0:
            return k
    raise ValueError(per_w)


def _round_up(v, m):
    return (v + m - 1) // m * m


def _sc_degree(dstp, n):
    nck = dstp.shape[1]
    ck = dstp.shape[2]
    rpt = _round_up(-(-n // NS), 8)
    np_ = rpt * NS
    win = 4
    mesh = plsc.VectorSubcoreMesh(core_axis_name="c", subcore_axis_name="s")

    def body(dstp_h, zeros_h, ones_h, out_h, idx_v, ones_v, acc_sh, sem):
        c = lax.axis_index("c")
        s = lax.axis_index("s")
        wid = s * NC + c

        base_r = s * rpt
        pltpu.sync_copy(zeros_h, acc_sh.at[pl.ds(base_r, rpt)])
        pltpu.sync_copy(ones_h, ones_v)
        pltpu.sync_copy(dstp_h.at[wid], idx_v)
        plsc.subcore_barrier()

        def swait():
            pltpu.make_async_copy(ones_v, acc_sh.at[pl.ds(0, ck)], sem).wait()

        def step(i, _):
            pltpu.async_copy(ones_v, acc_sh.at[idx_v.at[i]], sem, add=True)

            @pl.when(i >= win)
            def _():
                swait()
            return 0
        lax.fori_loop(0, nck, step, 0)
        for _ in range(min(win, nck)):
            swait()
        plsc.subcore_barrier()
        pltpu.sync_copy(acc_sh.at[pl.ds(base_r, rpt)],
                        out_h.at[pl.ds(c * np_ + base_r, rpt)])

    return pl.kernel(
        body,
        out_type=jax.ShapeDtypeStruct((NC * np_, D), jnp.float32),
        mesh=mesh,
        scratch_types=[
            pltpu.VMEM((nck, ck), jnp.int32),
            pltpu.VMEM((ck, D), jnp.float32),
            pltpu.VMEM_SHARED((np_, D), jnp.float32),
            pltpu.SemaphoreType.DMA,
        ],
    )(dstp, jnp.zeros((rpt, D), jnp.float32),
      jnp.ones((ck, D), jnp.float32)), np_


def _sc_scatter(table, srcp, dstp):
    n = table.shape[0]
    nck = srcp.shape[1]
    ck = srcp.shape[2]
    rpt = _round_up(-(-n // NS), 8)
    np_ = rpt * NS
    PH = 2
    L = -(-nck // PH)
    mesh = plsc.VectorSubcoreMesh(core_axis_name="c", subcore_axis_name="s")

    def body(table_h, src_h, dst_h, zeros_h, out_h, idx_s, idx_d,
             rows_a, rows_b, acc_sh, gsem_a, gsem_b):
        c = lax.axis_index("c")
        s = lax.axis_index("s")
        wid = s * NC + c

        base_r = s * rpt
        pltpu.sync_copy(zeros_h, acc_sh.at[pl.ds(base_r, rpt)])

        def gather(i, buf, sem):
            pltpu.async_copy(table_h.at[idx_s.at[i]], buf, sem)

        def gwait(buf, sem):
            pltpu.make_async_copy(table_h.at[pl.ds(0, ck)], buf, sem).wait()

        for p in range(PH):
            cbase = p * L
            lp = min(L, nck - cbase)
            if lp <= 0:
                continue
            pltpu.sync_copy(src_h.at[wid, pl.ds(cbase, lp)],
                            idx_s.at[pl.ds(0, lp)])
            pltpu.sync_copy(dst_h.at[wid, pl.ds(cbase, lp)],
                            idx_d.at[pl.ds(0, lp)])

            gather(0, rows_a, gsem_a)
            if lp > 1:
                gather(1, rows_b, gsem_b)
            if p == 0:
                plsc.subcore_barrier()

            def pair(k, _):
                i = 2 * k
                gwait(rows_a, gsem_a)
                pltpu.sync_copy(rows_a, acc_sh.at[idx_d.at[i]], add=True)

                @pl.when(i + 2 < lp)
                def _():
                    gather(i + 2, rows_a, gsem_a)
                gwait(rows_b, gsem_b)
                pltpu.sync_copy(rows_b, acc_sh.at[idx_d.at[i + 1]], add=True)

                @pl.when(i + 3 < lp)
                def _():
                    gather(i + 3, rows_b, gsem_b)
                return 0
            lax.fori_loop(0, lp // 2, pair, 0)
            if lp % 2:
                gwait(rows_a, gsem_a)
                pltpu.sync_copy(rows_a, acc_sh.at[idx_d.at[lp - 1]], add=True)

        plsc.subcore_barrier()
        pltpu.sync_copy(acc_sh.at[pl.ds(base_r, rpt)],
                        out_h.at[pl.ds(c * np_ + base_r, rpt)])

    return pl.kernel(
        body,
        out_type=jax.ShapeDtypeStruct((NC * np_, D), jnp.float32),
        mesh=mesh,
        scratch_types=[
            pltpu.VMEM((L, ck), jnp.int32),
            pltpu.VMEM((L, ck), jnp.int32),
            pltpu.VMEM((ck, D), jnp.float32),
            pltpu.VMEM((ck, D), jnp.float32),
            pltpu.VMEM_SHARED((np_, D), jnp.float32),
            pltpu.SemaphoreType.DMA,
            pltpu.SemaphoreType.DMA,
        ],
    )(table, srcp, dstp, jnp.zeros((rpt, D), jnp.float32)), np_


def _tc_stage1(xp, w1, degp):
    np_ = xp.shape[0]
    bn = np_ // 16

    def body(x_r, w_r, d0_r, d1_r, h_r, hh_r, dinv_r):
        deg = d0_r[...] + d1_r[...] + 1.0
        dinv = lax.rsqrt(deg)
        h = jnp.dot(x_r[...], w_r[...], preferred_element_type=jnp.float32)
        h_r[...] = h
        hh_r[...] = h * dinv
        dinv_r[...] = dinv

    return pl.pallas_call(
        body,
        grid=(16,),
        in_specs=[
            pl.BlockSpec((bn, D), lambda i: (i, 0)),
            pl.BlockSpec((D, D), lambda i: (0, 0)),
            pl.BlockSpec((bn, D), lambda i: (i, 0)),
            pl.BlockSpec((bn, D), lambda i: (i + 16, 0)),
        ],
        out_specs=[
            pl.BlockSpec((bn, D), lambda i: (i, 0)),
            pl.BlockSpec((bn, D), lambda i: (i, 0)),
            pl.BlockSpec((bn, D), lambda i: (i, 0)),
        ],
        out_shape=[
            jax.ShapeDtypeStruct((np_, D), jnp.float32),
            jax.ShapeDtypeStruct((np_, D), jnp.float32),
            jax.ShapeDtypeStruct((np_, D), jnp.float32),
        ],
    )(xp, w1, degp, degp)


def _tc_stage2(aggp, h1, dinv, w2, b1):
    np_ = h1.shape[0]
    bn = np_ // 16

    def body(a0_r, a1_r, h1_r, dinv_r, w_r, b_r, h2_r, hh2_r):
        dinv = dinv_r[...]
        t = (a0_r[...] + a1_r[...]) * dinv + h1_r[...] * (dinv * dinv) + b_r[...]
        t = jnp.maximum(t, 0.0)
        h2 = jnp.dot(t, w_r[...], preferred_element_type=jnp.float32)
        h2_r[...] = h2
        hh2_r[...] = h2 * dinv

    return pl.pallas_call(
        body,
        grid=(16,),
        in_specs=[
            pl.BlockSpec((bn, D), lambda i: (i, 0)),
            pl.BlockSpec((bn, D), lambda i: (i + 16, 0)),
            pl.BlockSpec((bn, D), lambda i: (i, 0)),
            pl.BlockSpec((bn, D), lambda i: (i, 0)),
            pl.BlockSpec((D, D), lambda i: (0, 0)),
            pl.BlockSpec((1, D), lambda i: (0, 0)),
        ],
        out_specs=[
            pl.BlockSpec((bn, D), lambda i: (i, 0)),
            pl.BlockSpec((bn, D), lambda i: (i, 0)),
        ],
        out_shape=[
            jax.ShapeDtypeStruct((np_, D), jnp.float32),
            jax.ShapeDtypeStruct((np_, D), jnp.float32),
        ],
    )(aggp, aggp, h1, dinv, w2, b1.reshape(1, D))


def _tc_stage3(aggp, h2, dinv, b2):
    np_ = h2.shape[0]
    bn = np_ // 16

    def body(a0_r, a1_r, h2_r, dinv_r, b_r, o_r):
        dinv = dinv_r[...]
        o_r[...] = ((a0_r[...] + a1_r[...]) * dinv
                    + h2_r[...] * (dinv * dinv) + b_r[...])

    return pl.pallas_call(
        body,
        grid=(16,),
        in_specs=[
            pl.BlockSpec((bn, D), lambda i: (i, 0)),
            pl.BlockSpec((bn, D), lambda i: (i + 16, 0)),
            pl.BlockSpec((bn, D), lambda i: (i, 0)),
            pl.BlockSpec((bn, D), lambda i: (i, 0)),
            pl.BlockSpec((1, D), lambda i: (0, 0)),
        ],
        out_specs=pl.BlockSpec((bn, D), lambda i: (i, 0)),
        out_shape=jax.ShapeDtypeStruct((np_, D), jnp.float32),
    )(aggp, aggp, h2, dinv, b2.reshape(1, D))


def kernel(x, edge_index, W1, b1, W2, b2):
    n = x.shape[0]
    e = edge_index.shape[1]
    per_w = e // NW
    ck = 128
    nck = -(-per_w // ck)
    pad = nck * ck - per_w
    srcp = jnp.pad(edge_index[0].reshape(NW, per_w),
                   ((0, 0), (0, pad))).reshape(NW, nck, ck)
    dstp = jnp.pad(edge_index[1].reshape(NW, per_w), ((0, 0), (0, pad)),
                   constant_values=n).reshape(NW, nck, ck)
    np_ = _round_up(-(-n // NS), 8) * NS
    xp = jnp.pad(x, ((0, np_ - n), (0, 0)))

    degp, _ = _sc_degree(dstp, n)
    h1, hh1, dinv = _tc_stage1(xp, W1, degp)
    agg1, _ = _sc_scatter(hh1, srcp, dstp)
    h2, hh2 = _tc_stage2(agg1, h1, dinv, W2, b1)
    agg2, _ = _sc_scatter(hh2, srcp, dstp)
    return _tc_stage3(agg2, h2, dinv, b2)[:n]


# speedup vs baseline: 1.1196x; 1.0010x over previous
import jax
import jax.numpy as jnp
from jax import lax
from jax.experimental import pallas as pl
from jax.experimental.pallas import tpu as pltpu
from jax.experimental.pallas import tpu_sc as plsc

NC = 2
NS = 16
NW = NC * NS
D = 128


def _round_up(v, m):
    return (v + m - 1) // m * m


def _sc_degree(dstp, n):
    nck = dstp.shape[1]
    ck = dstp.shape[2]
    rpt = _round_up(-(-n // NS), 8)
    np_ = rpt * NS
    win = 4
    mesh = plsc.VectorSubcoreMesh(core_axis_name="c", subcore_axis_name="s")

    def body(dstp_h, zeros_h, ones_h, out_h, idx_v, ones_v, acc_sh, sem):
        c = lax.axis_index("c")
        s = lax.axis_index("s")
        wid = s * NC + c

        base_r = s * rpt
        pltpu.sync_copy(zeros_h, acc_sh.at[pl.ds(base_r, rpt)])
        pltpu.sync_copy(ones_h, ones_v)
        pltpu.sync_copy(dstp_h.at[wid], idx_v)
        plsc.subcore_barrier()

        def swait():
            pltpu.make_async_copy(ones_v, acc_sh.at[pl.ds(0, ck)], sem).wait()

        def step(i, _):
            pltpu.async_copy(ones_v, acc_sh.at[idx_v.at[i]], sem, add=True)

            @pl.when(i >= win)
            def _():
                swait()
            return 0
        lax.fori_loop(0, nck, step, 0)
        for _ in range(min(win, nck)):
            swait()
        plsc.subcore_barrier()
        pltpu.sync_copy(acc_sh.at[pl.ds(base_r, rpt)],
                        out_h.at[pl.ds(c * np_ + base_r, rpt)])

    return pl.kernel(
        body,
        out_type=jax.ShapeDtypeStruct((NC * np_, D), jnp.float32),
        mesh=mesh,
        scratch_types=[
            pltpu.VMEM((nck, ck), jnp.int32),
            pltpu.VMEM((ck, D), jnp.float32),
            pltpu.VMEM_SHARED((np_, D), jnp.float32),
            pltpu.SemaphoreType.DMA,
        ],
    )(dstp, jnp.zeros((rpt, D), jnp.float32),
      jnp.ones((ck, D), jnp.float32)), np_


def _sc_scatter(table, srcp, dstp):
    n = table.shape[0]
    nck = srcp.shape[1]
    ck = srcp.shape[2]
    rpt = _round_up(-(-n // NS), 8)
    np_ = rpt * NS
    PH = 2
    L = -(-nck // PH)
    mesh = plsc.VectorSubcoreMesh(core_axis_name="c", subcore_axis_name="s")

    def body(table_h, src_h, dst_h, zeros_h, out_h, idx_s, idx_d,
             rows_a, rows_b, acc_sh, gsem_a, gsem_b):
        c = lax.axis_index("c")
        s = lax.axis_index("s")
        wid = s * NC + c

        base_r = s * rpt
        pltpu.sync_copy(zeros_h, acc_sh.at[pl.ds(base_r, rpt)])

        def gather(i, buf, sem):
            pltpu.async_copy(table_h.at[idx_s.at[i]], buf, sem)

        def gwait(buf, sem):
            pltpu.make_async_copy(table_h.at[pl.ds(0, ck)], buf, sem).wait()

        for p in range(PH):
            cbase = p * L
            lp = min(L, nck - cbase)
            if lp <= 0:
                continue
            pltpu.sync_copy(src_h.at[wid, pl.ds(cbase, lp)],
                            idx_s.at[pl.ds(0, lp)])
            pltpu.sync_copy(dst_h.at[wid, pl.ds(cbase, lp)],
                            idx_d.at[pl.ds(0, lp)])

            gather(0, rows_a, gsem_a)
            if lp > 1:
                gather(1, rows_b, gsem_b)
            if p == 0:
                plsc.subcore_barrier()

            def pair(k, _):
                i = 2 * k
                gwait(rows_a, gsem_a)
                pltpu.sync_copy(rows_a, acc_sh.at[idx_d.at[i]], add=True)

                @pl.when(i + 2 < lp)
                def _():
                    gather(i + 2, rows_a, gsem_a)
                gwait(rows_b, gsem_b)
                pltpu.sync_copy(rows_b, acc_sh.at[idx_d.at[i + 1]], add=True)

                @pl.when(i + 3 < lp)
                def _():
                    gather(i + 3, rows_b, gsem_b)
                return 0
            lax.fori_loop(0, lp // 2, pair, 0)
            if lp % 2:
                gwait(rows_a, gsem_a)
                pltpu.sync_copy(rows_a, acc_sh.at[idx_d.at[lp - 1]], add=True)

        plsc.subcore_barrier()
        pltpu.sync_copy(acc_sh.at[pl.ds(base_r, rpt)],
                        out_h.at[pl.ds(c * np_ + base_r, rpt)])

    return pl.kernel(
        body,
        out_type=jax.ShapeDtypeStruct((NC * np_, D), jnp.float32),
        mesh=mesh,
        scratch_types=[
            pltpu.VMEM((L, ck), jnp.int32),
            pltpu.VMEM((L, ck), jnp.int32),
            pltpu.VMEM((ck, D), jnp.float32),
            pltpu.VMEM((ck, D), jnp.float32),
            pltpu.VMEM_SHARED((np_, D), jnp.float32),
            pltpu.SemaphoreType.DMA,
            pltpu.SemaphoreType.DMA,
        ],
    )(table, srcp, dstp, jnp.zeros((rpt, D), jnp.float32)), np_


def _tc_stage1(xp, w1, degp):
    np_ = xp.shape[0]
    bn = np_ // 16

    def body(x_r, w_r, d0_r, d1_r, h_r, hh_r, dinv_r):
        deg = d0_r[...] + d1_r[...] + 1.0
        dinv = lax.rsqrt(deg)
        h = jnp.dot(x_r[...], w_r[...], preferred_element_type=jnp.float32)
        h_r[...] = h
        hh_r[...] = h * dinv
        dinv_r[...] = dinv

    return pl.pallas_call(
        body,
        grid=(16,),
        in_specs=[
            pl.BlockSpec((bn, D), lambda i: (i, 0)),
            pl.BlockSpec((D, D), lambda i: (0, 0)),
            pl.BlockSpec((bn, D), lambda i: (i, 0)),
            pl.BlockSpec((bn, D), lambda i: (i + 16, 0)),
        ],
        out_specs=[
            pl.BlockSpec((bn, D), lambda i: (i, 0)),
            pl.BlockSpec((bn, D), lambda i: (i, 0)),
            pl.BlockSpec((bn, D), lambda i: (i, 0)),
        ],
        out_shape=[
            jax.ShapeDtypeStruct((np_, D), jnp.float32),
            jax.ShapeDtypeStruct((np_, D), jnp.float32),
            jax.ShapeDtypeStruct((np_, D), jnp.float32),
        ],
    )(xp, w1, degp, degp)


def _tc_stage2(aggp, h1, dinv, w2, b1):
    np_ = h1.shape[0]
    bn = np_ // 16

    def body(a0_r, a1_r, h1_r, dinv_r, w_r, b_r, h2_r, hh2_r):
        dinv = dinv_r[...]
        t = (a0_r[...] + a1_r[...]) * dinv + h1_r[...] * (dinv * dinv) + b_r[...]
        t = jnp.maximum(t, 0.0)
        h2 = jnp.dot(t, w_r[...], preferred_element_type=jnp.float32)
        h2_r[...] = h2
        hh2_r[...] = h2 * dinv

    return pl.pallas_call(
        body,
        grid=(16,),
        in_specs=[
            pl.BlockSpec((bn, D), lambda i: (i, 0)),
            pl.BlockSpec((bn, D), lambda i: (i + 16, 0)),
            pl.BlockSpec((bn, D), lambda i: (i, 0)),
            pl.BlockSpec((bn, D), lambda i: (i, 0)),
            pl.BlockSpec((D, D), lambda i: (0, 0)),
            pl.BlockSpec((1, D), lambda i: (0, 0)),
        ],
        out_specs=[
            pl.BlockSpec((bn, D), lambda i: (i, 0)),
            pl.BlockSpec((bn, D), lambda i: (i, 0)),
        ],
        out_shape=[
            jax.ShapeDtypeStruct((np_, D), jnp.float32),
            jax.ShapeDtypeStruct((np_, D), jnp.float32),
        ],
    )(aggp, aggp, h1, dinv, w2, b1.reshape(1, D))


def _tc_stage3(aggp, h2, dinv, b2):
    np_ = h2.shape[0]
    bn = np_ // 16

    def body(a0_r, a1_r, h2_r, dinv_r, b_r, o_r):
        dinv = dinv_r[...]
        o_r[...] = ((a0_r[...] + a1_r[...]) * dinv
                    + h2_r[...] * (dinv * dinv) + b_r[...])

    return pl.pallas_call(
        body,
        grid=(16,),
        in_specs=[
            pl.BlockSpec((bn, D), lambda i: (i, 0)),
            pl.BlockSpec((bn, D), lambda i: (i + 16, 0)),
            pl.BlockSpec((bn, D), lambda i: (i, 0)),
            pl.BlockSpec((bn, D), lambda i: (i, 0)),
            pl.BlockSpec((1, D), lambda i: (0, 0)),
        ],
        out_specs=pl.BlockSpec((bn, D), lambda i: (i, 0)),
        out_shape=jax.ShapeDtypeStruct((np_, D), jnp.float32),
    )(aggp, aggp, h2, dinv, b2.reshape(1, D))


def kernel(x, edge_index, W1, b1, W2, b2):
    n = x.shape[0]
    e = edge_index.shape[1]
    per_w = e // NW
    ck = 128
    nck = -(-per_w // ck)
    pad = nck * ck - per_w
    srcp = jnp.pad(edge_index[0].reshape(NW, per_w),
                   ((0, 0), (0, pad))).reshape(NW, nck, ck)
    dstp = jnp.pad(edge_index[1].reshape(NW, per_w), ((0, 0), (0, pad)),
                   constant_values=n).reshape(NW, nck, ck)
    np_ = _round_up(-(-n // NS), 8) * NS
    xp = jnp.pad(x, ((0, np_ - n), (0, 0)))

    degp, _ = _sc_degree(dstp, n)
    h1, hh1, dinv = _tc_stage1(xp, W1, degp)
    agg1, _ = _sc_scatter(hh1, srcp, dstp)
    h2, hh2 = _tc_stage2(agg1, h1, dinv, W2, b1)
    agg2, _ = _sc_scatter(hh2, srcp, dstp)
    return _tc_stage3(agg2, h2, dinv, b2)[:n]
